# Initial kernel scaffold; baseline (speedup 1.0000x reference)
import jax
import jax.numpy as jnp
from jax.experimental import pallas as pl


def kernel(x, lut):
    raise NotImplementedError("write your pallas kernel here")


# speedup vs baseline: 1.2921x; 1.2921x over previous
import functools
import math

import jax
import jax.numpy as jnp
from jax import lax
from jax.experimental import pallas as pl
from jax.experimental.pallas import tpu as pltpu
from jax.experimental.pallas import tpu_sc as plsc

_EMBED = 32
_SCALE = math.sqrt(float(_EMBED))
_LANES = 16
_NUM_WORKERS = 32
_CHUNK = 1024


def kernel(x, lut):
    batch, seq = x.shape
    n = batch * seq
    embed = lut.shape[1]
    idx = x.reshape(n).astype(jnp.int32)

    per_w = n // _NUM_WORKERS
    n_chunks = per_w // _CHUNK

    mesh = plsc.VectorSubcoreMesh(core_axis_name="c", subcore_axis_name="s")

    @functools.partial(
        pl.kernel,
        mesh=mesh,
        out_type=jax.ShapeDtypeStruct((n, embed), jnp.float32),
        scratch_types=[
            pltpu.VMEM((_CHUNK,), jnp.int32),
            pltpu.VMEM((_CHUNK, embed), jnp.float32),
            pltpu.SemaphoreType.DMA,
        ],
        compiler_params=pltpu.CompilerParams(use_tc_tiling_on_sc=False),
    )
    def sc_gather_scale(idx_hbm, table_hbm, out_hbm, idx_v, rows_v, sem):
        wid = lax.axis_index("s") * 2 + lax.axis_index("c")
        base = wid * per_w

        @pl.loop(0, n_chunks)
        def _(c):
            off = base + c * _CHUNK
            pltpu.sync_copy(idx_hbm.at[pl.ds(off, _CHUNK)], idx_v)
            pltpu.async_copy(table_hbm.at[idx_v], rows_v, sem).wait()

            @pl.loop(0, _CHUNK)
            def _(r):
                @pl.loop(0, embed, step=_LANES)
                def _(col):
                    slc = (pl.ds(r, 1), pl.ds(col, _LANES))
                    rows_v.at[slc][...] = rows_v.at[slc][...] * _SCALE

            pltpu.sync_copy(rows_v, out_hbm.at[pl.ds(off, _CHUNK)])

    out = sc_gather_scale(idx, lut)
    return out.reshape(batch, seq, embed)


# speedup vs baseline: 1.4744x; 1.1410x over previous
import functools
import math

import jax
import jax.numpy as jnp
from jax import lax
from jax.experimental import pallas as pl
from jax.experimental.pallas import tpu as pltpu
from jax.experimental.pallas import tpu_sc as plsc

_EMBED = 32
_SCALE = math.sqrt(float(_EMBED))
_LANES = 16
_NUM_WORKERS = 32
_CHUNK = 1600


def kernel(x, lut):
    batch, seq = x.shape
    n = batch * seq
    embed = lut.shape[1]
    idx = x.reshape(n).astype(jnp.int32)

    per_w = n // _NUM_WORKERS
    n_chunks = per_w // _CHUNK
    assert n_chunks % 2 == 0 and n_chunks * _CHUNK == per_w

    mesh = plsc.VectorSubcoreMesh(core_axis_name="c", subcore_axis_name="s")

    @functools.partial(
        pl.kernel,
        mesh=mesh,
        out_type=jax.ShapeDtypeStruct((n, embed), jnp.float32),
        scratch_types=[
            pltpu.VMEM((_CHUNK,), jnp.int32),
            pltpu.VMEM((_CHUNK,), jnp.int32),
            pltpu.VMEM((_CHUNK, embed), jnp.float32),
            pltpu.VMEM((_CHUNK, embed), jnp.float32),
            pltpu.SemaphoreType.DMA,
            pltpu.SemaphoreType.DMA,
            pltpu.SemaphoreType.DMA,
            pltpu.SemaphoreType.DMA,
        ],
        compiler_params=pltpu.CompilerParams(use_tc_tiling_on_sc=False),
    )
    def sc_gather_scale(idx_hbm, table_hbm, out_hbm,
                        idx0, idx1, rows0, rows1,
                        gsem0, gsem1, ssem0, ssem1):
        wid = lax.axis_index("s") * 2 + lax.axis_index("c")
        base = wid * per_w

        def start(c, idx_b, rows_b, gsem_b):
            off = base + c * _CHUNK
            pltpu.sync_copy(idx_hbm.at[pl.ds(off, _CHUNK)], idx_b)
            pltpu.async_copy(table_hbm.at[idx_b], rows_b, gsem_b)

        def gather_wait(idx_b, rows_b, gsem_b):
            pltpu.make_async_copy(table_hbm.at[idx_b], rows_b, gsem_b).wait()

        def store_wait(rows_b, ssem_b):
            pltpu.make_async_copy(
                rows_b, out_hbm.at[pl.ds(0, _CHUNK)], ssem_b).wait()

        def scale(rows_b):
            @pl.loop(0, _CHUNK, step=8)
            def _(r):
                for dr in range(8):
                    for col in range(0, embed, _LANES):
                        slc = (pl.ds(r + dr, 1), pl.ds(col, _LANES))
                        rows_b.at[slc][...] = rows_b.at[slc][...] * _SCALE

        start(0, idx0, rows0, gsem0)

        @pl.loop(0, n_chunks, step=2)
        def _(c):
            @pl.when(c > 0)
            def _():
                store_wait(rows1, ssem1)

            start(c + 1, idx1, rows1, gsem1)

            gather_wait(idx0, rows0, gsem0)
            scale(rows0)
            pltpu.async_copy(
                rows0, out_hbm.at[pl.ds(base + c * _CHUNK, _CHUNK)], ssem0)

            @pl.when(c + 2 < n_chunks)
            def _():
                store_wait(rows0, ssem0)
                start(c + 2, idx0, rows0, gsem0)

            gather_wait(idx1, rows1, gsem1)
            scale(rows1)
            pltpu.async_copy(
                rows1, out_hbm.at[pl.ds(base + (c + 1) * _CHUNK, _CHUNK)],
                ssem1)

        store_wait(rows0, ssem0)
        store_wait(rows1, ssem1)

    out = sc_gather_scale(idx, lut)
    return out.reshape(batch, seq, embed)


# speedup vs baseline: 1.5728x; 1.0668x over previous
import functools
import math

import jax
import jax.numpy as jnp
from jax import lax
from jax.experimental import pallas as pl
from jax.experimental.pallas import tpu as pltpu
from jax.experimental.pallas import tpu_sc as plsc

_EMBED = 32
_SCALE = math.sqrt(float(_EMBED))
_NUM_WORKERS = 32
_CHUNK = 1600
_PACK_C = 2048
_Q = _PACK_C // 4


def _pack_body(in_ref, out_ref):
    t = (in_ref[...] * _SCALE).T
    out_ref[...] = jnp.concatenate(
        [t[0 * _Q:1 * _Q], t[1 * _Q:2 * _Q],
         t[2 * _Q:3 * _Q], t[3 * _Q:4 * _Q]], axis=1)


def kernel(x, lut):
    batch, seq = x.shape
    n = batch * seq
    vocab, embed = lut.shape
    idx = x.reshape(n).astype(jnp.int32)

    lut_t = jnp.transpose(lut)
    grid = (vocab + _PACK_C - 1) // _PACK_C
    vocab_pad = grid * _PACK_C
    lut_packed = pl.pallas_call(
        _pack_body,
        grid=(grid,),
        in_specs=[pl.BlockSpec((embed, _PACK_C), lambda i: (0, i))],
        out_specs=pl.BlockSpec((_Q, 128), lambda i: (i, 0)),
        out_shape=jax.ShapeDtypeStruct((vocab_pad * embed // 128, 128),
                                       jnp.float32),
        compiler_params=pltpu.CompilerParams(
            dimension_semantics=("parallel",)),
    )(lut_t)
    lut_rows = lut_packed.reshape(vocab_pad, embed)

    per_w = n // _NUM_WORKERS
    n_chunks = per_w // _CHUNK
    assert n_chunks % 2 == 0 and n_chunks * _CHUNK == per_w

    mesh = plsc.VectorSubcoreMesh(core_axis_name="c", subcore_axis_name="s")

    @functools.partial(
        pl.kernel,
        mesh=mesh,
        out_type=jax.ShapeDtypeStruct((n, embed), jnp.float32),
        scratch_types=[
            pltpu.VMEM((_CHUNK,), jnp.int32),
            pltpu.VMEM((_CHUNK,), jnp.int32),
            pltpu.VMEM((_CHUNK, embed), jnp.float32),
            pltpu.VMEM((_CHUNK, embed), jnp.float32),
            pltpu.SemaphoreType.DMA,
            pltpu.SemaphoreType.DMA,
            pltpu.SemaphoreType.DMA,
            pltpu.SemaphoreType.DMA,
        ],
        compiler_params=pltpu.CompilerParams(use_tc_tiling_on_sc=False),
    )
    def sc_gather(idx_hbm, table_hbm, out_hbm,
                  idx0, idx1, rows0, rows1,
                  gsem0, gsem1, ssem0, ssem1):
        wid = lax.axis_index("s") * 2 + lax.axis_index("c")
        base = wid * per_w

        def start(c, idx_b, rows_b, gsem_b):
            off = base + c * _CHUNK
            pltpu.sync_copy(idx_hbm.at[pl.ds(off, _CHUNK)], idx_b)

            @pl.loop(0, _CHUNK, step=16)
            def _(k):
                v = idx_b.at[pl.ds(k, 16)][...]
                rho = (((v >> 11) << 11) + ((v & (_Q - 1)) << 2)
                       + ((v >> 9) & 3))
                idx_b.at[pl.ds(k, 16)][...] = rho

            pltpu.async_copy(table_hbm.at[idx_b], rows_b, gsem_b)

        def gather_wait(idx_b, rows_b, gsem_b):
            pltpu.make_async_copy(table_hbm.at[idx_b], rows_b, gsem_b).wait()

        def store_wait(rows_b, ssem_b):
            pltpu.make_async_copy(
                rows_b, out_hbm.at[pl.ds(0, _CHUNK)], ssem_b).wait()

        start(0, idx0, rows0, gsem0)

        @pl.loop(0, n_chunks, step=2)
        def _(c):
            @pl.when(c > 0)
            def _():
                store_wait(rows1, ssem1)

            start(c + 1, idx1, rows1, gsem1)

            gather_wait(idx0, rows0, gsem0)
            pltpu.async_copy(
                rows0, out_hbm.at[pl.ds(base + c * _CHUNK, _CHUNK)], ssem0)

            @pl.when(c + 2 < n_chunks)
            def _():
                store_wait(rows0, ssem0)
                start(c + 2, idx0, rows0, gsem0)

            gather_wait(idx1, rows1, gsem1)
            pltpu.async_copy(
                rows1, out_hbm.at[pl.ds(base + (c + 1) * _CHUNK, _CHUNK)],
                ssem1)

        store_wait(rows0, ssem0)
        store_wait(rows1, ssem1)

    out = sc_gather(idx, lut_rows)
    return out.reshape(batch, seq, embed)
